# speedup vs baseline: 202.4298x; 202.4298x over previous
import functools

import jax
import jax.numpy as jnp
from jax.experimental import pallas as pl
from jax.experimental.pallas import tpu as pltpu

_LANES = 128


def _nms_match_kernel(x1_ref, y1_ref, x2_ref, y2_ref, area_ref, score_ref,
                      thr_ref, match_ref, *, rows, total):
    iou_thr = thr_ref[0, 0]
    score_thr = thr_ref[0, 1]

    x1 = x1_ref[...]
    y1 = y1_ref[...]
    x2 = x2_ref[...]
    y2 = y2_ref[...]
    area = area_ref[...]

    idx = (jax.lax.broadcasted_iota(jnp.int32, (rows, _LANES), 0) * _LANES
           + jax.lax.broadcasted_iota(jnp.int32, (rows, _LANES), 1))

    valid = score_ref[...] > score_thr
    match_ref[...] = jnp.where(valid, jnp.int32(-1), jnp.int32(-2))

    lane_iota = jax.lax.broadcasted_iota(jnp.int32, (1, _LANES), 1)

    def first_unmatched():
        cand = jnp.where(match_ref[...] == -1, idx, jnp.int32(total))
        return jnp.min(cand)

    def body(r):
        rr = r // _LANES
        rc = r % _LANES

        def extract(ref):
            row = ref[pl.ds(rr, 1), :]
            return jnp.sum(jnp.where(lane_iota == rc, row, 0.0))

        bx1 = extract(x1_ref)
        by1 = extract(y1_ref)
        bx2 = extract(x2_ref)
        by2 = extract(y2_ref)
        barea = extract(area_ref)

        ix1 = jnp.maximum(bx1, x1)
        iy1 = jnp.maximum(by1, y1)
        ix2 = jnp.minimum(bx2, x2)
        iy2 = jnp.minimum(by2, y2)
        inter = (jnp.maximum(ix2 - ix1, 0.0) * jnp.maximum(iy2 - iy1, 0.0))
        iou = inter / (barea + area - inter + 1e-10)

        m = match_ref[...]
        members = (m == -1) & (iou > iou_thr)
        m = jnp.where(members, r, m)
        match_ref[...] = m
        cand = jnp.where(m == -1, idx, jnp.int32(total))
        return jnp.min(cand)

    jax.lax.while_loop(lambda r: r < total, body, first_unmatched())


def _nms_match_sorted(sx1, sy1, sx2, sy2, sarea, sscore, thr, rows):
    total = rows * _LANES
    kern = functools.partial(_nms_match_kernel, rows=rows, total=total)
    return pl.pallas_call(
        kern,
        out_shape=jax.ShapeDtypeStruct((rows, _LANES), jnp.int32),
    )(sx1, sy1, sx2, sy2, sarea, sscore, thr)


def kernel(boxes, scores, iou_threshold, score_threshold):
    B, C, n = scores.shape
    n_idx = jnp.arange(n, dtype=jnp.int32)
    rows = -(-n // _LANES)
    total = rows * _LANES
    pad = total - n

    thr = jnp.stack([iou_threshold[0], score_threshold[0]]).reshape(1, 2)

    out_rows = []
    for b in range(B):
        bx = boxes[b]
        for c in range(C):
            sc = scores[b, c]
            order = jnp.argsort(-sc)
            sb = bx[order]
            ss = sc[order]
            sx1 = jnp.pad(sb[:, 0], (0, pad)).reshape(rows, _LANES)
            sy1 = jnp.pad(sb[:, 1], (0, pad)).reshape(rows, _LANES)
            sx2 = jnp.pad(sb[:, 2], (0, pad)).reshape(rows, _LANES)
            sy2 = jnp.pad(sb[:, 3], (0, pad)).reshape(rows, _LANES)
            sarea = (sx2 - sx1) * (sy2 - sy1)
            ss_p = jnp.pad(ss, (0, pad), constant_values=-jnp.inf).reshape(
                rows, _LANES)

            msorted = _nms_match_sorted(sx1, sy1, sx2, sy2, sarea, ss_p, thr,
                                        rows).reshape(-1)[:n]

            keeper_orig = jnp.where(
                msorted >= 0,
                order[jnp.clip(msorted, 0, n - 1)].astype(jnp.int32),
                msorted)
            match_to = jnp.zeros(n, jnp.int32).at[order].set(keeper_orig)

            valid_pair = (match_to >= 0) & (match_to != n_idx)
            rows_out = jnp.stack([
                jnp.full((n,), b, dtype=jnp.int32),
                jnp.full((n,), c, dtype=jnp.int32),
                match_to,
                n_idx,
            ], axis=1)
            out_rows.append(
                jnp.where(valid_pair[:, None], rows_out, jnp.int32(-1)))
    return jnp.concatenate(out_rows, axis=0)


# speedup vs baseline: 299.8912x; 1.4815x over previous
import functools

import jax
import jax.numpy as jnp
from jax.experimental import pallas as pl
from jax.experimental.pallas import tpu as pltpu

_LANES = 128


def _nms_match_kernel(box_smem, x1_ref, y1_ref, x2_ref, y2_ref, area_ref,
                      score_ref, thr_ref, match_ref, *, rows, total):
    iou_thr = thr_ref[0, 0]
    score_thr = thr_ref[0, 1]

    x1 = x1_ref[...]
    y1 = y1_ref[...]
    x2 = x2_ref[...]
    y2 = y2_ref[...]
    area = area_ref[...]

    idx = (jax.lax.broadcasted_iota(jnp.int32, (rows, _LANES), 0) * _LANES
           + jax.lax.broadcasted_iota(jnp.int32, (rows, _LANES), 1))
    idx_f = idx.astype(jnp.float32)
    big_f = jnp.float32(total)

    valid = score_ref[...] > score_thr
    match_ref[...] = jnp.where(valid, jnp.int32(-1), jnp.int32(-2))

    def first_unmatched():
        cand = jnp.where(match_ref[...] == -1, idx_f, big_f)
        return jnp.min(cand).astype(jnp.int32)

    def body(r):
        r4 = r * 4
        bx1 = box_smem[r4]
        by1 = box_smem[r4 + 1]
        bx2 = box_smem[r4 + 2]
        by2 = box_smem[r4 + 3]
        barea = (bx2 - bx1) * (by2 - by1)

        ix1 = jnp.maximum(bx1, x1)
        iy1 = jnp.maximum(by1, y1)
        ix2 = jnp.minimum(bx2, x2)
        iy2 = jnp.minimum(by2, y2)
        inter = (jnp.maximum(ix2 - ix1, 0.0) * jnp.maximum(iy2 - iy1, 0.0))
        iou = inter / (barea + area - inter + 1e-10)

        m = match_ref[...]
        members = (m == -1) & (iou > iou_thr)
        m = jnp.where(members, r, m)
        match_ref[...] = m
        cand = jnp.where(m == -1, idx_f, big_f)
        return jnp.min(cand).astype(jnp.int32)

    jax.lax.while_loop(lambda r: r < total, body, first_unmatched())


def _nms_match_sorted(box_packed, sx1, sy1, sx2, sy2, sarea, sscore, thr,
                      rows):
    total = rows * _LANES
    kern = functools.partial(_nms_match_kernel, rows=rows, total=total)
    return pl.pallas_call(
        kern,
        in_specs=[
            pl.BlockSpec(memory_space=pltpu.SMEM),
            pl.BlockSpec(memory_space=pltpu.VMEM),
            pl.BlockSpec(memory_space=pltpu.VMEM),
            pl.BlockSpec(memory_space=pltpu.VMEM),
            pl.BlockSpec(memory_space=pltpu.VMEM),
            pl.BlockSpec(memory_space=pltpu.VMEM),
            pl.BlockSpec(memory_space=pltpu.VMEM),
            pl.BlockSpec(memory_space=pltpu.SMEM),
        ],
        out_shape=jax.ShapeDtypeStruct((rows, _LANES), jnp.int32),
    )(box_packed, sx1, sy1, sx2, sy2, sarea, sscore, thr)


def kernel(boxes, scores, iou_threshold, score_threshold):
    B, C, n = scores.shape
    n_idx = jnp.arange(n, dtype=jnp.int32)
    rows = -(-n // _LANES)
    total = rows * _LANES
    pad = total - n

    thr = jnp.stack([iou_threshold[0], score_threshold[0]]).reshape(1, 2)

    out_rows = []
    for b in range(B):
        bx = boxes[b]
        for c in range(C):
            sc = scores[b, c]
            order = jnp.argsort(-sc)
            sb = bx[order]
            ss = sc[order]
            sx1 = jnp.pad(sb[:, 0], (0, pad)).reshape(rows, _LANES)
            sy1 = jnp.pad(sb[:, 1], (0, pad)).reshape(rows, _LANES)
            sx2 = jnp.pad(sb[:, 2], (0, pad)).reshape(rows, _LANES)
            sy2 = jnp.pad(sb[:, 3], (0, pad)).reshape(rows, _LANES)
            sarea = (sx2 - sx1) * (sy2 - sy1)
            ss_p = jnp.pad(ss, (0, pad), constant_values=-jnp.inf).reshape(
                rows, _LANES)
            box_packed = jnp.pad(sb, ((0, pad), (0, 0))).reshape(-1)

            msorted = _nms_match_sorted(box_packed, sx1, sy1, sx2, sy2, sarea,
                                        ss_p, thr, rows).reshape(-1)[:n]

            keeper_orig = jnp.where(
                msorted >= 0,
                order[jnp.clip(msorted, 0, n - 1)].astype(jnp.int32),
                msorted)
            match_to = jnp.zeros(n, jnp.int32).at[order].set(keeper_orig)

            valid_pair = (match_to >= 0) & (match_to != n_idx)
            rows_out = jnp.stack([
                jnp.full((n,), b, dtype=jnp.int32),
                jnp.full((n,), c, dtype=jnp.int32),
                match_to,
                n_idx,
            ], axis=1)
            out_rows.append(
                jnp.where(valid_pair[:, None], rows_out, jnp.int32(-1)))
    return jnp.concatenate(out_rows, axis=0)
